# baseline (speedup 1.0000x reference)
import functools

import numpy as np
import jax
import jax.numpy as jnp
from jax import lax
from jax.experimental import pallas as pl
from jax.experimental.pallas import tpu as pltpu
from jax.experimental.pallas import tpu_sc as plsc

R = 128
V = 100000
K = 50
CAP = 512
CW = 128
NCH = 782
PV = NCH * CW
CMW = 784
HC = 128
INT_MIN = np.int32(-2147483648)
NEG_INF = np.float32(-np.inf)
TINY = np.float32(1.1754943508222875e-38)


def _monokey(x):
    b = lax.bitcast_convert_type(x, jnp.int32)
    return jnp.where(b >= 0, b, b ^ jnp.int32(0x7FFFFFFF))


def _inv_monokey(k):
    b = jnp.where(k >= 0, k, k ^ jnp.int32(0x7FFFFFFF))
    return lax.bitcast_convert_type(b, jnp.float32)


def _kth_largest_key(keys, k):
    nonneg = jnp.sum(jnp.where(keys >= 0, 1, 0).astype(jnp.int32), axis=1,
                     keepdims=True)
    t = jnp.where(nonneg >= k, jnp.int32(0), INT_MIN)

    def body(i, t):
        cand = t | (jnp.int32(1) << (jnp.int32(30) - i))
        cnt = jnp.sum(jnp.where(keys >= cand, 1, 0).astype(jnp.int32),
                      axis=1, keepdims=True)
        return jnp.where(cnt >= k, cand, t)

    return lax.fori_loop(0, 31, body, t)


def _stage_a_kernel(x_ref, xpad_ref, cmax_ref, t0_ref):
    xp = jnp.concatenate(
        [x_ref[...], jnp.full((8, PV - V), NEG_INF, jnp.float32)], axis=1
    ).reshape(8, NCH, CW)
    xpad_ref[...] = xp
    cm = jnp.max(xp, axis=2)
    cmax_ref[...] = jnp.concatenate(
        [cm, jnp.full((8, CMW - NCH), NEG_INF, jnp.float32)], axis=1)

    keys = _monokey(cm)
    t = _kth_largest_key(keys, K)
    t0 = _inv_monokey(t - 4)
    t0_ref[...] = jnp.broadcast_to(t0, (8, 16))


def _stage_a(logits):
    return pl.pallas_call(
        _stage_a_kernel,
        grid=(R // 8,),
        in_specs=[pl.BlockSpec((8, V), lambda i: (i, 0))],
        out_specs=[pl.BlockSpec((8, NCH, CW), lambda i: (i, 0, 0)),
                   pl.BlockSpec((8, CMW), lambda i: (i, 0)),
                   pl.BlockSpec((8, 16), lambda i: (i, 0))],
        out_shape=[jax.ShapeDtypeStruct((R, NCH, CW), jnp.float32),
                   jax.ShapeDtypeStruct((R, CMW), jnp.float32),
                   jax.ShapeDtypeStruct((R, 16), jnp.float32)],
    )(logits)


@functools.cache
def _build_stage_b():
    mesh = plsc.VectorSubcoreMesh(core_axis_name="c", subcore_axis_name="s")
    return functools.partial(
        pl.kernel,
        mesh=mesh,
        compiler_params=pltpu.CompilerParams(needs_layout_passes=False),
        out_type=[
            jax.ShapeDtypeStruct((R * CAP,), jnp.int32),
            jax.ShapeDtypeStruct((R * CAP,), jnp.float32),
            jax.ShapeDtypeStruct((R * 16,), jnp.int32),
        ],
        scratch_types=[
            pltpu.VMEM((4 * CMW,), jnp.float32),
            pltpu.VMEM((4 * HC,), jnp.int32),
            pltpu.VMEM((4 * HC, CW), jnp.float32),
            pltpu.VMEM((4 * CAP,), jnp.int32),
            pltpu.VMEM((4 * CAP,), jnp.float32),
            pltpu.VMEM((64,), jnp.float32),
            pltpu.VMEM((64,), jnp.int32),
            pltpu.SemaphoreType.DMA,
            pltpu.SemaphoreType.DMA,
        ],
    )(_stage_b_body)


def _stage_b_body(chunks_hbm, flatpad_hbm, cmax_hbm, t0_hbm,
                  idx_hbm, val_hbm, cnt_hbm,
                  cm_v, hit_v, chk_v, idx_v, val_v, t0_v, cnt_v, sem0, semg):
    wid = lax.axis_index("s") * 2 + lax.axis_index("c")
    row0 = wid * 4
    iota = lax.iota(jnp.int32, 16)

    pltpu.sync_copy(t0_hbm.at[pl.ds(pl.multiple_of(row0 * 16, 8), 64)], t0_v)
    pltpu.sync_copy(cmax_hbm.at[pl.ds(pl.multiple_of(row0 * CMW, 8),
                                      4 * CMW)], cm_v)
    tvecs = [t0_v[pl.ds(rl * 16, 16)] for rl in range(4)]
    idx_rows = [idx_v.at[pl.ds(rl * CAP, CAP)] for rl in range(4)]
    hit_rows = [hit_v.at[pl.ds(rl * HC, HC)] for rl in range(4)]

    def zbody(z, _):
        idx_v[pl.ds(z * 16, 16)] = jnp.zeros((16,), jnp.int32)
        return 0
    lax.fori_loop(0, 4 * CAP // 16, zbody, 0)

    def zbody2(z, _):
        hit_v[pl.ds(z * 16, 16)] = jnp.zeros((16,), jnp.int32)
        return 0
    lax.fori_loop(0, 4 * HC // 16, zbody2, 0)

    hoffs = (jnp.full((16,), -1, jnp.int32),) * 4

    with jax.named_scope("b_phase1"):
        @plsc.parallel_loop(0, CMW, 16, unroll=2, carry=hoffs)
        def hscan(v, hoffs):
            new = []
            for rl in range(4):
                x = cm_v[pl.ds(rl * CMW + v, 16)]
                mask = x >= tvecs[rl]
                mi = jnp.where(mask, 1, 0).astype(jnp.int32)
                cum = plsc.cumsum(mi)
                pc = plsc.all_reduce_population_count(mask)
                gid = iota + ((row0 + rl) * NCH + v)
                tgt = hoffs[rl] + cum
                smask = jnp.logical_and(mask, tgt < HC)
                plsc.store_scatter(hit_rows[rl], [tgt], gid, mask=smask)
                new.append(hoffs[rl] + pc)
            return tuple(new)

        hoffs = hscan

    offs = []
    for rl in range(4):
        nh = jnp.minimum(jnp.max(hoffs[rl]) + 1, HC)
        nb = (nh + 31) // 32
        with jax.named_scope("b_chunkgather"):
            for b in range(HC // 32):
                @pl.when(b < nb)
                def _():
                    pltpu.async_copy(
                        chunks_hbm.at[hit_v.at[pl.ds(rl * HC + b * 32, 32)]],
                        chk_v.at[pl.ds(rl * HC + b * 32, 32)], semg)
            for b in range(HC // 32):
                @pl.when(b < nb)
                def _():
                    pltpu.make_async_copy(
                        chunks_hbm.at[hit_v.at[pl.ds(rl * HC, 32)]],
                        chk_v.at[pl.ds(rl * HC, 32)], semg).wait()

        def cscan(h, off, rl=rl):
            hsplat = jnp.zeros((16,), jnp.int32) + (rl * HC + h)
            gid = plsc.load_gather(hit_v, [hsplat])
            base = gid * CW
            for p in range(CW // 16):
                x = plsc.load_gather(chk_v, [hsplat, iota + p * 16])
                mask = x >= tvecs[rl]
                mi = jnp.where(mask, 1, 0).astype(jnp.int32)
                cum = plsc.cumsum(mi)
                pc = plsc.all_reduce_population_count(mask)
                idxv = base + (iota + p * 16)
                tgt = off + cum
                smask = jnp.logical_and(mask, tgt < CAP)
                plsc.store_scatter(idx_rows[rl], [tgt], idxv, mask=smask)
                off = off + pc
            return off

        with jax.named_scope("b_cscan"):
            off = lax.fori_loop(0, nh, cscan, jnp.full((16,), -1, jnp.int32))
        offs.append(off)

    with jax.named_scope("b_valgather"):
        for rl in range(4):
            cnt_v[pl.ds(rl * 16, 16)] = jnp.minimum(offs[rl] + 1, CAP)
            for g in range(CAP // 128):
                pltpu.async_copy(
                    flatpad_hbm.at[idx_v.at[pl.ds(rl * CAP + g * 128, 128)]],
                    val_v.at[pl.ds(rl * CAP + g * 128, 128)], semg).wait()

    pltpu.sync_copy(idx_v, idx_hbm.at[pl.ds(pl.multiple_of(row0 * CAP, 8),
                                            4 * CAP)])
    pltpu.sync_copy(val_v, val_hbm.at[pl.ds(pl.multiple_of(row0 * CAP, 8),
                                            4 * CAP)])
    pltpu.sync_copy(cnt_v, cnt_hbm.at[pl.ds(pl.multiple_of(row0 * 16, 8), 64)])


def _threefry_bits(x1u):
    k0 = jnp.uint32(0)
    k1 = jnp.uint32(42)
    k2 = k0 ^ k1 ^ jnp.uint32(0x1BD11BDA)
    ks = (k0, k1, k2)
    x0 = jnp.zeros_like(x1u) + k0
    x1 = x1u + k1
    rots = ((13, 15, 26, 6), (17, 29, 16, 24))
    for i in range(5):
        for rr in rots[i % 2]:
            x0 = x0 + x1
            x1 = (x1 << rr) | (x1 >> (32 - rr))
            x1 = x1 ^ x0
        x0 = x0 + ks[(i + 1) % 3]
        x1 = x1 + ks[(i + 2) % 3] + jnp.uint32(i + 1)
    return x0 ^ x1


def _stage_c_kernel(val_ref, idx_ref, cnt_ref, temp_ref, out_ref):
    vals = val_ref[...]
    idx = idx_ref[...]
    cnt = cnt_ref[...][:, 0:1]
    temp = temp_ref[...]

    colj = lax.broadcasted_iota(jnp.int32, (R, CAP), 1)
    rowi = lax.broadcasted_iota(jnp.int32, (R, CAP), 0)
    valid = colj < cnt

    l = vals / temp
    keys = jnp.where(valid, _monokey(l), INT_MIN)
    kth = _kth_largest_key(keys, K)
    keep = keys >= kth

    m = jnp.max(jnp.where(keep, l, NEG_INF), axis=1, keepdims=True)
    e = jnp.where(keep, jnp.exp(l - m), jnp.float32(0.0))
    s = jnp.sum(e, axis=1, keepdims=True)
    lp = jnp.log(e / s + jnp.float32(1e-30))

    rng_idx = idx - rowi * jnp.int32(PV - V)
    bits = _threefry_bits(rng_idx.astype(jnp.uint32))
    f = lax.bitcast_convert_type(
        (bits >> jnp.uint32(9)) | jnp.uint32(0x3F800000),
        jnp.float32) - jnp.float32(1.0)
    u = jnp.maximum(TINY, f + TINY)
    g = -jnp.log(-jnp.log(u))

    score = jnp.where(keep, lp + g, NEG_INF)
    best = jnp.max(score, axis=1, keepdims=True)
    wincol = jnp.min(jnp.where(score == best, colj, jnp.int32(2 ** 30)),
                     axis=1, keepdims=True)
    vocab_idx = idx - rowi * jnp.int32(PV)
    token = jnp.sum(jnp.where(colj == wincol, vocab_idx, 0), axis=1,
                    keepdims=True)
    out_ref[...] = token


def _stage_c(vals, idx, cnts, temps):
    return pl.pallas_call(
        _stage_c_kernel,
        in_specs=[pl.BlockSpec((R, CAP), lambda: (0, 0)),
                  pl.BlockSpec((R, CAP), lambda: (0, 0)),
                  pl.BlockSpec((R, 16), lambda: (0, 0)),
                  pl.BlockSpec((R, 1), lambda: (0, 0))],
        out_specs=pl.BlockSpec((R, 1), lambda: (0, 0)),
        out_shape=jax.ShapeDtypeStruct((R, 1), jnp.int32),
    )(vals, idx, cnts, temps)


def kernel(logits, temperatures, top_k=50):
    del top_k
    logits = logits.astype(jnp.float32)
    xpad, cmax, t0 = _stage_a(logits)
    idx, vals, cnts = _build_stage_b()(
        xpad.reshape(R * NCH, CW), xpad.reshape(-1),
        cmax.reshape(-1), t0.reshape(-1))
    tok = _stage_c(vals.reshape(R, CAP), idx.reshape(R, CAP),
                   cnts.reshape(R, 16),
                   temperatures.astype(jnp.float32).reshape(R, 1))
    return tok.reshape(R)


# speedup vs baseline: 2.9300x; 2.9300x over previous
import functools

import numpy as np
import jax
import jax.numpy as jnp
from jax import lax
from jax.experimental import pallas as pl
from jax.experimental.pallas import tpu as pltpu
from jax.experimental.pallas import tpu_sc as plsc

R = 128
V = 100000
K = 50
CAP = 512
CW = 128
NCH = 782
PV = NCH * CW
CMW = 784
HC = 128
INT_MIN = np.int32(-2147483648)
NEG_INF = np.float32(-np.inf)
TINY = np.float32(1.1754943508222875e-38)


def _monokey(x):
    b = lax.bitcast_convert_type(x, jnp.int32)
    return jnp.where(b >= 0, b, b ^ jnp.int32(0x7FFFFFFF))


def _inv_monokey(k):
    b = jnp.where(k >= 0, k, k ^ jnp.int32(0x7FFFFFFF))
    return lax.bitcast_convert_type(b, jnp.float32)


def _kth_largest_key(keys, k):
    nonneg = jnp.sum(jnp.where(keys >= 0, 1, 0).astype(jnp.int32), axis=1,
                     keepdims=True)
    t = jnp.where(nonneg >= k, jnp.int32(0), INT_MIN)

    def body(i, t):
        cand = t | (jnp.int32(1) << (jnp.int32(30) - i))
        cnt = jnp.sum(jnp.where(keys >= cand, 1, 0).astype(jnp.int32),
                      axis=1, keepdims=True)
        return jnp.where(cnt >= k, cand, t)

    return lax.fori_loop(0, 31, body, t)


def _stage_a1_kernel(x_ref, xpad_ref, t0_ref):
    x = x_ref[...]
    xpad_ref[:, 0:V] = x
    xpad_ref[:, V:PV] = jnp.full((8, PV - V), NEG_INF, jnp.float32)

    nfull = V // 128
    accs = [x[:, s * 128:(s + 1) * 128] for s in range(8)]
    for s in range(8, nfull):
        accs[s % 8] = jnp.maximum(accs[s % 8], x[:, s * 128:(s + 1) * 128])
    tail = x[:, nfull * 128:V]
    accs[0] = jnp.maximum(
        accs[0],
        jnp.concatenate([tail, jnp.full((8, 96), NEG_INF, jnp.float32)], 1))
    m = accs[0]
    for s in range(1, 8):
        m = jnp.maximum(m, accs[s])

    keys = _monokey(m)
    t = _kth_largest_key(keys, K)
    t0 = _inv_monokey(t - 4)
    t0_ref[...] = jnp.broadcast_to(t0, (8, 16))


def _stage_a1(logits):
    return pl.pallas_call(
        _stage_a1_kernel,
        grid=(R // 8,),
        in_specs=[pl.BlockSpec((8, V), lambda i: (i, 0))],
        out_specs=[pl.BlockSpec((8, PV), lambda i: (i, 0)),
                   pl.BlockSpec((8, 16), lambda i: (i, 0))],
        out_shape=[jax.ShapeDtypeStruct((R, PV), jnp.float32),
                   jax.ShapeDtypeStruct((R, 16), jnp.float32)],
    )(logits)


def _stage_a2_kernel(xp_ref, cmax_ref):
    cm = jnp.max(xp_ref[...], axis=2)
    cmax_ref[...] = jnp.concatenate(
        [cm, jnp.full((8, CMW - NCH), NEG_INF, jnp.float32)], axis=1)


def _stage_a2(xpad3d):
    return pl.pallas_call(
        _stage_a2_kernel,
        grid=(R // 8,),
        in_specs=[pl.BlockSpec((8, NCH, CW), lambda i: (i, 0, 0))],
        out_specs=pl.BlockSpec((8, CMW), lambda i: (i, 0)),
        out_shape=jax.ShapeDtypeStruct((R, CMW), jnp.float32),
    )(xpad3d)


@functools.cache
def _build_stage_b():
    mesh = plsc.VectorSubcoreMesh(core_axis_name="c", subcore_axis_name="s")
    return functools.partial(
        pl.kernel,
        mesh=mesh,
        compiler_params=pltpu.CompilerParams(needs_layout_passes=False),
        out_type=[
            jax.ShapeDtypeStruct((R * CAP,), jnp.int32),
            jax.ShapeDtypeStruct((R * CAP,), jnp.float32),
            jax.ShapeDtypeStruct((R * 16,), jnp.int32),
        ],
        scratch_types=[
            pltpu.VMEM((4 * CMW,), jnp.float32),
            pltpu.VMEM((4 * HC,), jnp.int32),
            pltpu.VMEM((4 * HC, CW), jnp.float32),
            pltpu.VMEM((4 * CAP,), jnp.int32),
            pltpu.VMEM((4 * CAP,), jnp.float32),
            pltpu.VMEM((64,), jnp.float32),
            pltpu.VMEM((64,), jnp.int32),
            pltpu.SemaphoreType.DMA,
            pltpu.SemaphoreType.DMA,
        ],
    )(_stage_b_body)


def _stage_b_body(chunks_hbm, cmax_hbm, t0_hbm,
                  idx_hbm, val_hbm, cnt_hbm,
                  cm_v, hit_v, chk_v, idx_v, val_v, t0_v, cnt_v, sem0, semg):
    wid = lax.axis_index("s") * 2 + lax.axis_index("c")
    row0 = wid * 4
    iota = lax.iota(jnp.int32, 16)

    pltpu.sync_copy(t0_hbm.at[pl.ds(pl.multiple_of(row0 * 16, 8), 64)], t0_v)
    pltpu.sync_copy(cmax_hbm.at[pl.ds(pl.multiple_of(row0 * CMW, 8),
                                      4 * CMW)], cm_v)
    tvecs = [t0_v[pl.ds(rl * 16, 16)] for rl in range(4)]
    idx_rows = [idx_v.at[pl.ds(rl * CAP, CAP)] for rl in range(4)]
    val_rows = [val_v.at[pl.ds(rl * CAP, CAP)] for rl in range(4)]
    hit_rows = [hit_v.at[pl.ds(rl * HC, HC)] for rl in range(4)]

    def zbody(z, _):
        idx_v[pl.ds(z * 16, 16)] = jnp.zeros((16,), jnp.int32)
        return 0
    lax.fori_loop(0, 4 * CAP // 16, zbody, 0)

    def zbody2(z, _):
        hit_v[pl.ds(z * 16, 16)] = jnp.zeros((16,), jnp.int32)
        return 0
    lax.fori_loop(0, 4 * HC // 16, zbody2, 0)

    hoffs = (jnp.full((16,), -1, jnp.int32),) * 4

    with jax.named_scope("b_phase1"):
        @plsc.parallel_loop(0, CMW, 16, unroll=2, carry=hoffs)
        def hscan(v, hoffs):
            new = []
            for rl in range(4):
                x = cm_v[pl.ds(rl * CMW + v, 16)]
                mask = x >= tvecs[rl]
                mi = jnp.where(mask, 1, 0).astype(jnp.int32)
                cum = plsc.cumsum(mi)
                pc = plsc.all_reduce_population_count(mask)
                gid = iota + ((row0 + rl) * NCH + v)
                tgt = hoffs[rl] + cum
                smask = jnp.logical_and(mask, tgt < HC)
                plsc.store_scatter(hit_rows[rl], [tgt], gid, mask=smask)
                new.append(hoffs[rl] + pc)
            return tuple(new)

        hoffs = hscan

    offs = []
    for rl in range(4):
        nh = jnp.minimum(jnp.max(hoffs[rl]) + 1, HC)
        nb = (nh + 31) // 32
        with jax.named_scope("b_chunkgather"):
            for b in range(HC // 32):
                @pl.when(b < nb)
                def _():
                    pltpu.async_copy(
                        chunks_hbm.at[hit_v.at[pl.ds(rl * HC + b * 32, 32)]],
                        chk_v.at[pl.ds(rl * HC + b * 32, 32)], semg)
            for b in range(HC // 32):
                @pl.when(b < nb)
                def _():
                    pltpu.make_async_copy(
                        chunks_hbm.at[hit_v.at[pl.ds(rl * HC, 32)]],
                        chk_v.at[pl.ds(rl * HC, 32)], semg).wait()

        def cscan(h, off, rl=rl):
            hsplat = jnp.zeros((16,), jnp.int32) + (rl * HC + h)
            gid = plsc.load_gather(hit_v, [hsplat])
            base = gid * CW
            for p in range(CW // 16):
                x = plsc.load_gather(chk_v, [hsplat, iota + p * 16])
                mask = x >= tvecs[rl]
                mi = jnp.where(mask, 1, 0).astype(jnp.int32)
                cum = plsc.cumsum(mi)
                pc = plsc.all_reduce_population_count(mask)
                idxv = base + (iota + p * 16)
                tgt = off + cum
                smask = jnp.logical_and(mask, tgt < CAP)
                plsc.store_scatter(idx_rows[rl], [tgt], idxv, mask=smask)
                plsc.store_scatter(val_rows[rl], [tgt], x, mask=smask)
                off = off + pc
            return off

        with jax.named_scope("b_cscan"):
            off = lax.fori_loop(0, nh, cscan, jnp.full((16,), -1, jnp.int32))
        offs.append(off)

    for rl in range(4):
        cnt_v[pl.ds(rl * 16, 16)] = jnp.minimum(offs[rl] + 1, CAP)

    pltpu.sync_copy(idx_v, idx_hbm.at[pl.ds(pl.multiple_of(row0 * CAP, 8),
                                            4 * CAP)])
    pltpu.sync_copy(val_v, val_hbm.at[pl.ds(pl.multiple_of(row0 * CAP, 8),
                                            4 * CAP)])
    pltpu.sync_copy(cnt_v, cnt_hbm.at[pl.ds(pl.multiple_of(row0 * 16, 8), 64)])


def _threefry_bits(x1u):
    k0 = jnp.uint32(0)
    k1 = jnp.uint32(42)
    k2 = k0 ^ k1 ^ jnp.uint32(0x1BD11BDA)
    ks = (k0, k1, k2)
    x0 = jnp.zeros_like(x1u) + k0
    x1 = x1u + k1
    rots = ((13, 15, 26, 6), (17, 29, 16, 24))
    for i in range(5):
        for rr in rots[i % 2]:
            x0 = x0 + x1
            x1 = (x1 << rr) | (x1 >> (32 - rr))
            x1 = x1 ^ x0
        x0 = x0 + ks[(i + 1) % 3]
        x1 = x1 + ks[(i + 2) % 3] + jnp.uint32(i + 1)
    return x0 ^ x1


def _stage_c_kernel(val_ref, idx_ref, cnt_ref, temp_ref, out_ref):
    vals = val_ref[...]
    idx = idx_ref[...]
    cnt = cnt_ref[...][:, 0:1]
    temp = temp_ref[...]

    colj = lax.broadcasted_iota(jnp.int32, (R, CAP), 1)
    rowi = lax.broadcasted_iota(jnp.int32, (R, CAP), 0)
    valid = colj < cnt

    l = vals / temp
    keys = jnp.where(valid, _monokey(l), INT_MIN)
    kth = _kth_largest_key(keys, K)
    keep = keys >= kth

    m = jnp.max(jnp.where(keep, l, NEG_INF), axis=1, keepdims=True)
    e = jnp.where(keep, jnp.exp(l - m), jnp.float32(0.0))
    s = jnp.sum(e, axis=1, keepdims=True)
    lp = jnp.log(e / s + jnp.float32(1e-30))

    rng_idx = idx - rowi * jnp.int32(PV - V)
    bits = _threefry_bits(rng_idx.astype(jnp.uint32))
    f = lax.bitcast_convert_type(
        (bits >> jnp.uint32(9)) | jnp.uint32(0x3F800000),
        jnp.float32) - jnp.float32(1.0)
    u = jnp.maximum(TINY, f + TINY)
    g = -jnp.log(-jnp.log(u))

    score = jnp.where(keep, lp + g, NEG_INF)
    best = jnp.max(score, axis=1, keepdims=True)
    wincol = jnp.min(jnp.where(score == best, colj, jnp.int32(2 ** 30)),
                     axis=1, keepdims=True)
    vocab_idx = idx - rowi * jnp.int32(PV)
    token = jnp.sum(jnp.where(colj == wincol, vocab_idx, 0), axis=1,
                    keepdims=True)
    out_ref[...] = token


def _stage_c(vals, idx, cnts, temps):
    return pl.pallas_call(
        _stage_c_kernel,
        in_specs=[pl.BlockSpec((R, CAP), lambda: (0, 0)),
                  pl.BlockSpec((R, CAP), lambda: (0, 0)),
                  pl.BlockSpec((R, 16), lambda: (0, 0)),
                  pl.BlockSpec((R, 1), lambda: (0, 0))],
        out_specs=pl.BlockSpec((R, 1), lambda: (0, 0)),
        out_shape=jax.ShapeDtypeStruct((R, 1), jnp.int32),
    )(vals, idx, cnts, temps)


def kernel(logits, temperatures, top_k=50):
    del top_k
    logits = logits.astype(jnp.float32)
    xpad, t0 = _stage_a1(logits)
    cmax = _stage_a2(xpad.reshape(R, NCH, CW))
    idx, vals, cnts = _build_stage_b()(
        xpad.reshape(R * NCH, CW), cmax.reshape(-1), t0.reshape(-1))
    tok = _stage_c(vals.reshape(R, CAP), idx.reshape(R, CAP),
                   cnts.reshape(R, 16),
                   temperatures.astype(jnp.float32).reshape(R, 1))
    return tok.reshape(R)
